# baseline (speedup 1.0000x reference)
import jax
import jax.numpy as jnp
import numpy as np
from jax.experimental import pallas as pl

_LEVELS = [8, 8, 8, 5, 5, 5]
_NQ = 8
_EPS = 1e-3


def _fsq_tc_body(x_ref, win_ref, bin_ref, wout_ref, bout_ref, c_ref,
                 out_ref, idxT_ref):
    hl = c_ref[:, 0:1]
    off = c_ref[:, 1:2]
    shift = c_ref[:, 2:3]
    hw = c_ref[:, 3:4]
    basis = c_ref[:, 4:5]

    z = jax.lax.dot_general(
        x_ref[...], win_ref[...], (((1,), (0,)), ((), ())),
        preferred_element_type=jnp.float32)
    zT = z.T + bin_ref[...]

    r = jnp.tanh(zT + shift) * hl - off
    q = jnp.zeros_like(r)
    for i in range(_NQ):
        scale = c_ref[:, 5 + i:6 + i]
        zb = jnp.tanh(r / scale + shift) * hl - off
        rnd = jnp.round(zb)
        codes = rnd / hw
        idxf = jnp.sum((rnd + hw) * basis, axis=0)
        idxT_ref[i, :] = idxf.astype(jnp.int32)
        quant = codes * scale
        r = r - quant
        q = q + quant

    out_ref[...] = jax.lax.dot_general(
        q, wout_ref[...], (((0,), (0,)), ((), ())),
        preferred_element_type=jnp.float32) + bout_ref[...]


def kernel(x, W_in, b_in, W_out, b_out):
    B, N, D = x.shape
    T = B * N
    x2 = x.reshape(T, D)
    win8 = jnp.zeros((D, 8), jnp.float32).at[:, :6].set(W_in)
    bin8 = jnp.zeros((8, 1), jnp.float32).at[:6, 0].set(b_in)
    wout8 = jnp.zeros((8, D), jnp.float32).at[:6, :].set(W_out)
    bout2 = b_out.reshape(1, D)

    lev = jnp.array(_LEVELS + [2, 2], dtype=jnp.float32)
    half_l = (lev - 1.0) * (1.0 + _EPS) / 2.0
    offset = jnp.where(jnp.mod(lev, 2.0) == 0.0, 0.5, 0.0)
    shift = jnp.arctanh(offset / half_l)
    hw = jnp.floor(lev / 2.0)
    basis = jnp.concatenate([
        jnp.array(np.concatenate(([1], np.cumprod(_LEVELS[:-1]))),
                  dtype=jnp.float32),
        jnp.zeros((2,), jnp.float32)])
    cols = [half_l, offset, shift, hw, basis]
    cols += [(lev - 1.0) ** (-float(i)) for i in range(_NQ)]
    cols += [jnp.zeros((8,), jnp.float32)] * (24 - len(cols))
    consts = jnp.stack(cols, axis=1)

    BLK = 2048
    grid = (T // BLK,)
    out, idxT = pl.pallas_call(
        _fsq_tc_body,
        grid=grid,
        in_specs=[
            pl.BlockSpec((BLK, D), lambda i: (i, 0)),
            pl.BlockSpec((D, 8), lambda i: (0, 0)),
            pl.BlockSpec((8, 1), lambda i: (0, 0)),
            pl.BlockSpec((8, D), lambda i: (0, 0)),
            pl.BlockSpec((1, D), lambda i: (0, 0)),
            pl.BlockSpec((8, 24), lambda i: (0, 0)),
        ],
        out_specs=[
            pl.BlockSpec((BLK, D), lambda i: (i, 0)),
            pl.BlockSpec((8, BLK), lambda i: (0, i)),
        ],
        out_shape=[
            jax.ShapeDtypeStruct((T, D), jnp.float32),
            jax.ShapeDtypeStruct((8, T), jnp.int32),
        ],
    )(x2, win8, bin8, wout8, bout2, consts)

    indices = idxT.T.reshape(B, N, _NQ)
    return out.reshape(B, N, D), indices
